# baseline (speedup 1.0000x reference)
import functools

import jax
import jax.numpy as jnp
import numpy as np
from jax import lax
from jax.experimental import pallas as pl
from jax.experimental.pallas import tpu as pltpu
from jax.experimental.pallas import tpu_sc as plsc

N = 10000
E = 320000
D = 128
H = D // 2
NC, NS, L = 2, 16, 16

CK = 128
CHE = 160
EPT = CHE * CK
TPAD = EPT - E // NS
BCAP = 1664
RCH = 184
NSELF = 640
LROWS = 640
LSINK = 625
PADV = LSINK << 14
HSIZE = 10240
HSTRIDE = HSIZE // NS


_mesh = plsc.VectorSubcoreMesh(
    core_axis_name="c", subcore_axis_name="s", num_cores=NC, num_subcores=NS)
_sc_params = pltpu.CompilerParams(
    needs_layout_passes=False, use_tc_tiling_on_sc=False)


def _viota():
    return lax.iota(jnp.int32, L)


def _extract(ii, v, t):
    return jnp.sum(jnp.where(ii == t, v, 0))


@functools.partial(
    pl.kernel,
    out_type=(jax.ShapeDtypeStruct((NS, RCH, CK), jnp.int32),
              jax.ShapeDtypeStruct((NS * NS,), jnp.int32),
              jax.ShapeDtypeStruct((HSIZE,), jnp.int32)),
    mesh=_mesh,
    scratch_types=[
        pltpu.VMEM((CHE, CK), jnp.int32),
        pltpu.VMEM((CHE, CK), jnp.int32),
        pltpu.VMEM((NS, BCAP), jnp.int32),
        pltpu.VMEM((L,), jnp.int32),
        pltpu.VMEM((NS * NS,), jnp.int32),
        pltpu.VMEM((RCH, CK), jnp.int32),
        pltpu.VMEM((HSIZE,), jnp.int32),
        pltpu.VMEM((HSTRIDE,), jnp.int32),
        pltpu.VMEM((HSTRIDE,), jnp.int32),
        pltpu.VMEM_SHARED((NS * NS * BCAP,), jnp.int32),
        pltpu.VMEM_SHARED((NS * NS,), jnp.int32),
        pltpu.SemaphoreType.DMA,
    ],
    compiler_params=_sc_params,
)
def _route_kernel(src_hbm, dst_hbm, routed_hbm, cntm_hbm, deg_hbm,
                  sv, dv, bkt, cnt16, cntm_v, rlocal, hist, slab, red,
                  stage, cnts_sh, sem):
    c = lax.axis_index("c")
    s = lax.axis_index("s")

    @pl.when(c == 0)
    def _route():
        pltpu.sync_copy(src_hbm.at[s], sv)
        pltpu.sync_copy(dst_hbm.at[s], dv)
        ii = _viota()
        padv = jnp.full((L,), PADV, jnp.int32)

        for o in range(NS):
            def fill_body(i, _, o=o):
                bkt[o, pl.ds(pl.multiple_of(i * L, L), L)] = padv
                return 0
            lax.fori_loop(0, BCAP // L, fill_body, 0)
        cnt16[...] = jnp.zeros((L,), jnp.int32)

        ones_i = jnp.ones((L,), jnp.int32)
        prev_idx = jnp.maximum(ii - 1, 0)

        def bucket_body(j, _):
            for v in range(CK // L):
                sl = pl.ds(v * L, L)
                srcv = sv[j, sl]
                dstv = dv[j, sl]
                o = dstv & 15
                packed = ((dstv >> 4) << 14) | srcv
                o_s, p_s = plsc.sort_key_val(o, packed)
                prev = o_s.at[prev_idx].get(mode="promise_in_bounds")
                is_start = (prev != o_s) | (ii == 0)
                first = plsc.cummax(jnp.where(is_start, ii, 0))
                rank = ii - first
                base = plsc.load_gather(cnt16, [o_s])
                plsc.store_scatter(bkt, [o_s, base + rank], p_s)
                plsc.addupdate_scatter(cnt16, [o_s], ones_i)
            return 0
        lax.fori_loop(0, CHE, bucket_body, 0)

        pltpu.sync_copy(cnt16, cnts_sh.at[pl.ds(s * L, L)])
        pltpu.sync_copy(cnt16, cntm_hbm.at[pl.ds(s * L, L)])
        for o in range(NS):
            pltpu.sync_copy(
                bkt.at[o], stage.at[pl.ds((s * NS + o) * BCAP, BCAP)])
        plsc.subcore_barrier()

        pltpu.sync_copy(cnts_sh, cntm_v)
        cnt_col = plsc.load_gather(cntm_v, [ii * NS + s])
        offs = plsc.cumsum(cnt_col) - cnt_col + NSELF

        def rfill_body(i, _):
            rlocal[i >> 3, pl.ds(pl.multiple_of((i & 7) * L, L), L)] = padv
            return 0
        lax.fori_loop(0, RCH * CK // L, rfill_body, 0)

        def self_body(q, _):
            il = q * L + ii
            packed = (il << 14) | jnp.minimum(il * L + s, N - 1)
            plsc.store_scatter(rlocal, [il >> 7, il & 127], packed)
            return 0
        lax.fori_loop(0, NSELF // L, self_body, 0)

        for t in range(NS):
            pltpu.sync_copy(
                stage.at[pl.ds((t * NS + s) * BCAP, BCAP)], bkt.at[t])
        for t in range(NS):
            wt = _extract(ii, offs, t)
            trips = (_extract(ii, cnt_col, t) + (L - 1)) // L

            def cp_body(i, _, t=t, wt=wt):
                v = bkt[t, pl.ds(pl.multiple_of(i * L, L), L)]
                pos = wt + i * L + ii
                plsc.store_scatter(rlocal, [pos >> 7, pos & 127], v)
                return 0
            lax.fori_loop(0, trips, cp_body, 0)

        pltpu.sync_copy(rlocal, routed_hbm.at[s])

    @pl.when(c == 1)
    def _deg():
        pltpu.sync_copy(dst_hbm.at[s], dv)
        zeros = jnp.zeros((L,), jnp.int32)
        ones = jnp.ones((L,), jnp.int32)

        def zero_body(i, _):
            hist[pl.ds(pl.multiple_of(i * L, L), L)] = zeros
            return 0
        lax.fori_loop(0, HSIZE // L, zero_body, 0)

        def chunk_body(j, _):
            for v in range(CK // L):
                idx = dv[j, pl.ds(v * L, L)]
                plsc.addupdate_scatter(hist, [idx], ones)
            return 0
        lax.fori_loop(0, CHE, chunk_body, 0)

        pltpu.sync_copy(hist, stage.at[pl.ds(s * HSIZE, HSIZE)])
        plsc.subcore_barrier()
        base = pl.multiple_of(s * HSTRIDE, HSTRIDE)

        def add_body(i, _):
            o = pl.ds(pl.multiple_of(i * L, L), L)
            red[o] = red[o] + slab[o]
            return 0

        def cp_body(i, _):
            o = pl.ds(pl.multiple_of(i * L, L), L)
            red[o] = slab[o]
            return 0

        for t in range(NS):
            pltpu.sync_copy(
                stage.at[pl.ds(t * HSIZE + base, HSTRIDE)], slab)
            lax.fori_loop(0, HSTRIDE // L, cp_body if t == 0 else add_body, 0)
        pltpu.sync_copy(red, deg_hbm.at[pl.ds(base, HSTRIDE)])


def _agg_core(t_ref, out_ref, s, rlocal, cntm_v, ibuf, gb0, gb1, acc,
              gs0, gs1):
    ii = _viota()
    cnt_col = plsc.load_gather(cntm_v, [ii * NS + s])
    trips = (NSELF + jnp.sum(cnt_col) + (CK - 1)) // CK

    zeros = jnp.zeros((L,), jnp.float32)

    def zero_body(i, _):
        acc[i >> 2, pl.ds(pl.multiple_of((i & 3) * L, L), L)] = zeros
        return 0
    lax.fori_loop(0, LROWS * H // L, zero_body, 0)

    def prep_fire(j, ib, gb, gs):
        for k in range(CK // L):
            p = rlocal[j, pl.ds(k * L, L)]
            ibuf[ib, pl.ds(k * L, L)] = p & 0x3FFF
        pltpu.async_copy(t_ref.at[ibuf.at[ib]], gb, gs)

    def scatter(j, gb):
        def g_body(g, _):
            sl = pl.ds(pl.multiple_of(g * L, L), L)
            p = rlocal[j, sl]
            dl = p >> 14
            ev = g * L + ii
            for col in range(H):
                cv = ii * 0 + col
                val = plsc.load_gather(gb, [ev, cv])
                plsc.addupdate_scatter(acc, [dl, cv], val)
            return 0
        lax.fori_loop(0, CK // L, g_body, 0)

    def chunk_loop(j, _):
        for k in range(CK // L):
            p = rlocal[j, pl.ds(k * L, L)]
            ibuf[0, pl.ds(k * L, L)] = p & 0x3FFF
        pltpu.async_copy(t_ref.at[ibuf.at[0]], gb0, gs0).wait()
        scatter(j, gb0)
        return 0
    lax.fori_loop(0, trips, chunk_loop, 0)

    pltpu.sync_copy(acc.at[pl.ds(0, N // NS)], out_ref.at[s])


@functools.partial(
    pl.kernel,
    out_type=(jax.ShapeDtypeStruct((NS, N // NS, H), jnp.float32),
              jax.ShapeDtypeStruct((NS, N // NS, H), jnp.float32)),
    mesh=_mesh,
    scratch_types=[
        pltpu.VMEM((RCH, CK), jnp.int32),
        pltpu.VMEM((NS * NS,), jnp.int32),
        pltpu.VMEM((2, CK), jnp.int32),
        pltpu.VMEM((CK, H), jnp.float32),
        pltpu.VMEM((CK, H), jnp.float32),
        pltpu.VMEM((LROWS, H), jnp.float32),
        pltpu.SemaphoreType.DMA,
        pltpu.SemaphoreType.DMA,
    ],
    compiler_params=_sc_params,
)
def _agg_kernel(ta_hbm, tb_hbm, routed_hbm, cntm_hbm, outa, outb,
                rlocal, cntm_v, ibuf, gb0, gb1, acc, gs0, gs1):
    c = lax.axis_index("c")
    s = lax.axis_index("s")
    pltpu.sync_copy(routed_hbm.at[s], rlocal)
    pltpu.sync_copy(cntm_hbm, cntm_v)

    @pl.when(c == 0)
    def _():
        _agg_core(ta_hbm, outa, s, rlocal, cntm_v, ibuf, gb0, gb1, acc,
                  gs0, gs1)

    @pl.when(c == 1)
    def _():
        _agg_core(tb_hbm, outb, s, rlocal, cntm_v, ibuf, gb0, gb1, acc,
                  gs0, gs1)


def _unperm(a):
    return jnp.swapaxes(a, 0, 1).reshape(N, a.shape[-1])


def _mm1_body(x_ref, w_ref, deg_ref, ta_ref, tb_ref):
    dis = lax.rsqrt(deg_ref[...][:N].astype(jnp.float32) + 1.0)
    t = jnp.dot(x_ref[...], w_ref[...],
                preferred_element_type=jnp.float32) * dis
    ta_ref[...] = t[:, :H]
    tb_ref[...] = t[:, H:]


def _mid_body(aa_ref, ab_ref, deg_ref, b_ref, w_ref, ta_ref, tb_ref):
    dis = lax.rsqrt(deg_ref[...][:N].astype(jnp.float32) + 1.0)
    h = (jnp.concatenate([_unperm(aa_ref[...]), _unperm(ab_ref[...])], axis=1)
         * dis + b_ref[...])
    h = jnp.maximum(h, 0.0)
    t = jnp.dot(h, w_ref[...], preferred_element_type=jnp.float32) * dis
    ta_ref[...] = t[:, :H]
    tb_ref[...] = t[:, H:]


def _post_body(aa_ref, ab_ref, deg_ref, b_ref, o_ref):
    dis = lax.rsqrt(deg_ref[...][:N].astype(jnp.float32) + 1.0)
    o_ref[...] = (
        jnp.concatenate([_unperm(aa_ref[...]), _unperm(ab_ref[...])], axis=1)
        * dis + b_ref[...])


_half_pair = [jax.ShapeDtypeStruct((N, H), jnp.float32),
              jax.ShapeDtypeStruct((N, H), jnp.float32)]
_mm1 = pl.pallas_call(_mm1_body, out_shape=_half_pair)
_mid = pl.pallas_call(_mid_body, out_shape=_half_pair)
_post = pl.pallas_call(
    _post_body, out_shape=jax.ShapeDtypeStruct((N, D), jnp.float32))


def kernel(x, edge_index, W1, b1, W2, b2):
    src = edge_index[0].astype(jnp.int32).reshape(NS, E // NS)
    dst = edge_index[1].astype(jnp.int32).reshape(NS, E // NS)
    spad = jnp.zeros((NS, TPAD), jnp.int32)
    dpad = jnp.broadcast_to(
        N + (jnp.arange(TPAD, dtype=jnp.int32) % NS), (NS, TPAD))
    srcp = jnp.concatenate([src, spad], axis=1).reshape(NS, CHE, CK)
    dstp = jnp.concatenate([dst, dpad], axis=1).reshape(NS, CHE, CK)

    routed, cntm, deg = _route_kernel(srcp, dstp)
    deg = deg.reshape(HSIZE, 1)
    b1r = b1.reshape(1, D)
    b2r = b2.reshape(1, D)

    t1a, t1b = _mm1(x, W1, deg)
    a1a, a1b = _agg_kernel(t1a, t1b, routed, cntm)
    t2a, t2b = _mid(a1a, a1b, deg, b1r, W2)
    a2a, a2b = _agg_kernel(t2a, t2b, routed, cntm)
    return _post(a2a, a2b, deg, b2r)


# speedup vs baseline: 6.6338x; 6.6338x over previous
import functools

import jax
import jax.numpy as jnp
from jax import lax
from jax.experimental import pallas as pl
from jax.experimental.pallas import tpu as pltpu
from jax.experimental.pallas import tpu_sc as plsc

N = 10000
E = 320000
D = 128
H = D // 2
NC, NS, L = 2, 16, 16

CK = 128
CH = 157
EPT = CH * CK
EPAD = NS * EPT
SINK = N
ACCR = N + 16
HSIZE = 10240
HSTRIDE = HSIZE // NS

_mesh = plsc.VectorSubcoreMesh(
    core_axis_name="c", subcore_axis_name="s", num_cores=NC, num_subcores=NS)
_sc_params = pltpu.CompilerParams(
    needs_layout_passes=False, use_tc_tiling_on_sc=False)


@functools.partial(
    pl.kernel,
    out_type=jax.ShapeDtypeStruct((HSIZE,), jnp.float32),
    mesh=_mesh,
    scratch_types=[
        pltpu.VMEM((CH, CK), jnp.int32),
        pltpu.VMEM((HSIZE,), jnp.float32),
        pltpu.VMEM((HSTRIDE,), jnp.float32),
        pltpu.VMEM((HSTRIDE,), jnp.float32),
        pltpu.VMEM_SHARED((NS, HSIZE), jnp.float32),
        pltpu.SemaphoreType.DMA,
    ],
    compiler_params=_sc_params,
)
def _deg_kernel(dst_hbm, deg_hbm, dst_v, hist, slab, red, shared, sem):
    c = lax.axis_index("c")
    s = lax.axis_index("s")

    @pl.when(c == 0)
    def _():
        pltpu.sync_copy(dst_hbm.at[s], dst_v)
        zeros = jnp.zeros((L,), jnp.float32)
        ones = jnp.ones((L,), jnp.float32)

        def zero_body(i, _):
            hist[pl.ds(pl.multiple_of(i * L, L), L)] = zeros
            return 0
        lax.fori_loop(0, HSIZE // L, zero_body, 0)

        def chunk_body(j, _):
            for v in range(CK // L):
                idx = dst_v[j, pl.ds(v * L, L)]
                plsc.addupdate_scatter(hist, [idx], ones)
            return 0
        lax.fori_loop(0, CH, chunk_body, 0)

        pltpu.sync_copy(hist, shared.at[s])
        plsc.subcore_barrier()

        base = pl.multiple_of(s * HSTRIDE, HSTRIDE)

        def add_body(i, _):
            o = pl.ds(pl.multiple_of(i * L, L), L)
            red[o] = red[o] + slab[o]
            return 0

        def cp_body(i, _):
            o = pl.ds(pl.multiple_of(i * L, L), L)
            red[o] = slab[o]
            return 0

        for t in range(NS):
            pltpu.sync_copy(shared.at[t, pl.ds(base, HSTRIDE)], slab)
            lax.fori_loop(0, HSTRIDE // L, cp_body if t == 0 else add_body, 0)

        pltpu.sync_copy(red, deg_hbm.at[pl.ds(base, HSTRIDE)])


def _agg_core(t_ref, out_ref, s, src_v, dst_v, gbuf, acc, sem):
    rows = N // NS
    rbase = s * rows
    pltpu.sync_copy(t_ref.at[pl.ds(rbase, rows)], acc.at[pl.ds(rbase, rows)])
    plsc.subcore_barrier()

    def chunk_body(j, _):
        pltpu.async_copy(t_ref.at[src_v.at[j]], gbuf, sem).wait()
        pltpu.sync_copy(gbuf, acc.at[dst_v.at[j]], add=True)
        return 0
    lax.fori_loop(0, CH, chunk_body, 0)

    plsc.subcore_barrier()
    pltpu.sync_copy(acc.at[pl.ds(rbase, rows)], out_ref.at[pl.ds(rbase, rows)])


@functools.partial(
    pl.kernel,
    out_type=(jax.ShapeDtypeStruct((N, H), jnp.float32),
              jax.ShapeDtypeStruct((N, H), jnp.float32)),
    mesh=_mesh,
    scratch_types=[
        pltpu.VMEM((CH, CK), jnp.int32),
        pltpu.VMEM((CH, CK), jnp.int32),
        pltpu.VMEM((CK, H), jnp.float32),
        pltpu.VMEM_SHARED((ACCR, H), jnp.float32),
        pltpu.SemaphoreType.DMA,
    ],
    compiler_params=_sc_params,
)
def _agg_kernel(ta_hbm, tb_hbm, src_hbm, dst_hbm,
                outa, outb, src_v, dst_v, gbuf, acc, sem):
    c = lax.axis_index("c")
    s = lax.axis_index("s")
    pltpu.sync_copy(src_hbm.at[s], src_v)
    pltpu.sync_copy(dst_hbm.at[s], dst_v)

    @pl.when(c == 0)
    def _():
        _agg_core(ta_hbm, outa, s, src_v, dst_v, gbuf, acc, sem)

    @pl.when(c == 1)
    def _():
        _agg_core(tb_hbm, outb, s, src_v, dst_v, gbuf, acc, sem)


def _mm1_body(x_ref, w_ref, deg_ref, ta_ref, tb_ref):
    dis = lax.rsqrt(deg_ref[...][:N] + 1.0)
    t = jnp.dot(x_ref[...], w_ref[...],
                preferred_element_type=jnp.float32) * dis
    ta_ref[...] = t[:, :H]
    tb_ref[...] = t[:, H:]


def _mid_body(aa_ref, ab_ref, deg_ref, b_ref, w_ref, ta_ref, tb_ref):
    dis = lax.rsqrt(deg_ref[...][:N] + 1.0)
    h = jnp.concatenate([aa_ref[...], ab_ref[...]], axis=1) * dis + b_ref[...]
    h = jnp.maximum(h, 0.0)
    t = jnp.dot(h, w_ref[...], preferred_element_type=jnp.float32) * dis
    ta_ref[...] = t[:, :H]
    tb_ref[...] = t[:, H:]


def _post_body(aa_ref, ab_ref, deg_ref, b_ref, o_ref):
    dis = lax.rsqrt(deg_ref[...][:N] + 1.0)
    o_ref[...] = (jnp.concatenate([aa_ref[...], ab_ref[...]], axis=1) * dis
                  + b_ref[...])


_half_pair = [jax.ShapeDtypeStruct((N, H), jnp.float32),
              jax.ShapeDtypeStruct((N, H), jnp.float32)]
_mm1 = pl.pallas_call(_mm1_body, out_shape=_half_pair)
_mid = pl.pallas_call(_mid_body, out_shape=_half_pair)
_post = pl.pallas_call(
    _post_body, out_shape=jax.ShapeDtypeStruct((N, D), jnp.float32))


def kernel(x, edge_index, W1, b1, W2, b2):
    src = edge_index[0].astype(jnp.int32)
    dst = edge_index[1].astype(jnp.int32)
    pad = EPAD - E
    srcp = jnp.concatenate(
        [src, jnp.zeros((pad,), jnp.int32)]).reshape(NS, CH, CK)
    dstp = jnp.concatenate(
        [dst, jnp.full((pad,), SINK, jnp.int32)]).reshape(NS, CH, CK)

    deg = _deg_kernel(dstp).reshape(HSIZE, 1)
    b1r = b1.reshape(1, D)
    b2r = b2.reshape(1, D)

    t1a, t1b = _mm1(x, W1, deg)
    a1a, a1b = _agg_kernel(t1a, t1b, srcp, dstp)
    t2a, t2b = _mid(a1a, a1b, deg, b1r, W2)
    a2a, a2b = _agg_kernel(t2a, t2b, srcp, dstp)
    return _post(a2a, a2b, deg, b2r)
